# baseline (speedup 1.0000x reference)
import functools
import jax
import jax.numpy as jnp
from jax import lax
from jax.experimental import pallas as pl
from jax.experimental.pallas import tpu as pltpu
from jax.experimental.pallas import tpu_sc as plsc

N = 50000
E = 800000
NUM_REL = 3
NUM_GRAPHS = 64
HID = 64

N2 = 50176
EP = 802816
NP = 53248
MP_N = 53248
MP_G = 256
SINK_N = N
SINK_G = NUM_GRAPHS
CHUNK = 128
NW = 32


@functools.lru_cache(maxsize=None)
def _sc_scatter(nt, wd, ei, mp):
  per_w = ei // NW
  nch = per_w // CHUNK
  rpw = mp // 16
  cz = min(CHUNK, rpw)
  ncz = rpw // cz
  mesh = plsc.VectorSubcoreMesh(core_axis_name="c", subcore_axis_name="s")

  @functools.partial(
      pl.kernel, mesh=mesh,
      compiler_params=pltpu.CompilerParams(use_tc_tiling_on_sc=False),
      out_type=jax.ShapeDtypeStruct((2, mp, wd), jnp.float32),
      scratch_types=[
          pltpu.VMEM((CHUNK,), jnp.int32),
          pltpu.VMEM((CHUNK,), jnp.int32),
          pltpu.VMEM((CHUNK, wd), jnp.float32),
          pltpu.VMEM((CHUNK, wd), jnp.float32),
          pltpu.VMEM_SHARED((mp, wd), jnp.float32),
          pltpu.SemaphoreType.DMA,
      ],
  )
  def k(table, sidx, didx, out, sbuf, dbuf, rows, zbuf, shared, sem):
    c = lax.axis_index("c")
    s = lax.axis_index("s")
    w = s * 2 + c

    def zrow(i, carry):
      for kk in range(wd // 16):
        zbuf[i, pl.ds(kk * 16, 16)] = jnp.zeros((16,), jnp.float32)
      return carry
    lax.fori_loop(0, CHUNK, zrow, None)

    rbase = s * rpw
    def zcp(j, carry):
      pltpu.sync_copy(zbuf.at[pl.ds(0, cz)],
                      shared.at[pl.ds(rbase + j * cz, cz)])
      return carry
    lax.fori_loop(0, ncz, zcp, None)
    plsc.subcore_barrier()

    ebase = w * per_w
    def body(j, carry):
      b = ebase + j * CHUNK
      pltpu.sync_copy(sidx.at[pl.ds(b, CHUNK)], sbuf)
      pltpu.sync_copy(didx.at[pl.ds(b, CHUNK)], dbuf)
      pltpu.async_copy(table.at[sbuf], rows, sem).wait()
      pltpu.sync_copy(rows, shared.at[dbuf], add=True)
      return carry
    lax.fori_loop(0, nch, body, None)
    plsc.subcore_barrier()

    def cout(j, carry):
      pltpu.sync_copy(shared.at[pl.ds(rbase + j * cz, cz)],
                      out.at[c, pl.ds(rbase + j * cz, cz)])
      return carry
    lax.fori_loop(0, ncz, cout, None)

  return k


def _mask_body(dref, tref, oref):
  d = dref[...]
  t = tref[...]
  for r in range(NUM_REL):
    oref[r] = jnp.where(t == r, d, SINK_N)


def _masked_dst(dst2, typ2):
  eb = EP // 128
  return pl.pallas_call(
      _mask_body,
      grid=(eb // 8,),
      in_specs=[pl.BlockSpec((8, 128), lambda i: (i, 0)),
                pl.BlockSpec((8, 128), lambda i: (i, 0))],
      out_specs=pl.BlockSpec((NUM_REL, 8, 128), lambda i: (0, i, 0)),
      out_shape=jax.ShapeDtypeStruct((NUM_REL, eb, 128), jnp.int32),
  )(dst2, typ2)


def _embed_body(x0r, x1r, seref, ceref, w0ref, b0ref, oref):
  a = x0r[...]
  b = x1r[...]
  io8 = lax.broadcasted_iota(jnp.int32, (1024, 8), 1)
  s = (a == io8).astype(jnp.float32) @ seref[...]
  col = (b == io8).astype(jnp.float32) @ ceref[...]
  h16 = jnp.concatenate([s, col], axis=1)
  oref[...] = jnp.maximum(h16 @ w0ref[...] + b0ref[...], 0.0)


def _embed(x0, x1, shape_emb, color_emb, w0, b0):
  return pl.pallas_call(
      _embed_body,
      grid=(N2 // 1024,),
      in_specs=[pl.BlockSpec((1024, 1), lambda i: (i, 0)),
                pl.BlockSpec((1024, 1), lambda i: (i, 0)),
                pl.BlockSpec((8, 8), lambda i: (0, 0)),
                pl.BlockSpec((8, 8), lambda i: (0, 0)),
                pl.BlockSpec((16, HID), lambda i: (0, 0)),
                pl.BlockSpec((1, HID), lambda i: (0, 0))],
      out_specs=pl.BlockSpec((1024, HID), lambda i: (i, 0)),
      out_shape=jax.ShapeDtypeStruct((N2, HID), jnp.float32),
  )(x0, x1, shape_emb, color_emb, w0, b0.reshape(1, HID))


def _combine_body(href, sref, cref, wref, wrref, bref, oref):
  acc = href[...] @ wrref[...] + bref[...]
  for r in range(NUM_REL):
    sr = sref[r, 0] + sref[r, 1]
    cr = cref[r, 0] + cref[r, 1]
    acc = acc + (sr @ wref[r]) * (1.0 / jnp.maximum(cr, 1.0))
  oref[...] = jnp.maximum(acc, 0.0)


def _combine(h, s_all, cnt4, w, wr, b):
  return pl.pallas_call(
      _combine_body,
      grid=(N2 // 1024,),
      in_specs=[pl.BlockSpec((1024, HID), lambda i: (i, 0)),
                pl.BlockSpec((NUM_REL, 2, 1024, HID), lambda i: (0, 0, i, 0)),
                pl.BlockSpec((NUM_REL, 2, 1024, 1), lambda i: (0, 0, i, 0)),
                pl.BlockSpec((NUM_REL, HID, HID), lambda i: (0, 0, 0)),
                pl.BlockSpec((HID, HID), lambda i: (0, 0)),
                pl.BlockSpec((1, HID), lambda i: (0, 0))],
      out_specs=pl.BlockSpec((1024, HID), lambda i: (i, 0)),
      out_shape=jax.ShapeDtypeStruct((N2, HID), jnp.float32),
  )(h, s_all, cnt4, w, wr, b.reshape(1, HID))


def _final_body(pref, cref, wcref, bcref, oref):
  ps = pref[0, :NUM_GRAPHS, :] + pref[1, :NUM_GRAPHS, :]
  pc = cref[0, :NUM_GRAPHS, 0:1] + cref[1, :NUM_GRAPHS, 0:1]
  pooled = ps / jnp.maximum(pc, 1.0)
  oref[...] = pooled @ wcref[...] + bcref[...]


def _final(psum, pcnt, wc, bc):
  return pl.pallas_call(
      _final_body,
      out_shape=jax.ShapeDtypeStruct((NUM_GRAPHS, wc.shape[1]), jnp.float32),
  )(psum, pcnt, wc, bc.reshape(1, -1))


def _layer_scatter(h, src_p, dmask_flat):
  ha = h[:, :32]
  hb = h[:, 32:]
  k32 = _sc_scatter(N2, 32, EP, MP_N)
  parts = []
  for r in range(NUM_REL):
    sa = k32(ha, src_p, dmask_flat[r])
    sb = k32(hb, src_p, dmask_flat[r])
    parts.append(jnp.concatenate([sa, sb], axis=-1))
  return jnp.stack(parts)[:, :, :N2, :]


def kernel(x, edge_index, edge_type, batch, shape_emb, color_emb,
           W0, b0, W1, Wr1, b1, W2, Wr2, b2, Wc, bc):
  x = x.astype(jnp.int32)
  src = edge_index[0].astype(jnp.int32)
  dst = edge_index[1].astype(jnp.int32)
  typ = edge_type.astype(jnp.int32)
  batch = batch.astype(jnp.int32)

  x0 = jnp.pad(x[:, 0], (0, N2 - N)).reshape(N2, 1)
  x1 = jnp.pad(x[:, 1], (0, N2 - N)).reshape(N2, 1)
  src_p = jnp.pad(src, (0, EP - E))
  dst_p = jnp.pad(dst, (0, EP - E), constant_values=SINK_N)
  typ_p = jnp.pad(typ, (0, EP - E), constant_values=NUM_REL)

  dmask = _masked_dst(dst_p.reshape(EP // 128, 128),
                      typ_p.reshape(EP // 128, 128))
  dmask_flat = dmask.reshape(NUM_REL, EP)

  ones_tab = jnp.ones((8, 16), jnp.float32)
  zsrc = jnp.zeros((EP,), jnp.int32)
  k16 = _sc_scatter(8, 16, EP, MP_N)
  cnts = jnp.stack([k16(ones_tab, zsrc, dmask_flat[r]) for r in range(NUM_REL)])
  cnt4 = cnts[:, :, :N2, 0:1]

  h0 = _embed(x0, x1, shape_emb, color_emb, W0, b0)
  s1 = _layer_scatter(h0, src_p, dmask_flat)
  h1 = _combine(h0, s1, cnt4, W1, Wr1, b1)
  s2 = _layer_scatter(h1, src_p, dmask_flat)
  h2 = _combine(h1, s2, cnt4, W2, Wr2, b2)

  psrc = jnp.pad(jnp.arange(N, dtype=jnp.int32), (0, NP - N))
  pdst = jnp.pad(batch, (0, NP - N), constant_values=SINK_G)
  psum = _sc_scatter(N2, HID, NP, MP_G)(h2, psrc, pdst)
  pcnt = _sc_scatter(8, 16, NP, MP_G)(ones_tab, jnp.zeros((NP,), jnp.int32), pdst)
  return _final(psum, pcnt, Wc, bc)


# speedup vs baseline: 2.7773x; 2.7773x over previous
import functools
import jax
import jax.numpy as jnp
from jax import lax
from jax.experimental import pallas as pl
from jax.experimental.pallas import tpu as pltpu
from jax.experimental.pallas import tpu_sc as plsc

N = 50000
E = 800000
NUM_REL = 3
NUM_GRAPHS = 64
HID = 64

N2 = 50176
EP = 802816
NP = 53248
MP_N = 53248
MP_G = 256
SINK_N = N
SINK_G = NUM_GRAPHS
CHUNK = 128
NW = 32


@functools.lru_cache(maxsize=None)
def _sc_scatter(nt, wd, ei, mp, ones_src=False):
  per_w = ei // NW
  nch = per_w // CHUNK
  rpw = mp // 16
  cz = min(CHUNK, rpw)
  ncz = rpw // cz
  mesh = plsc.VectorSubcoreMesh(core_axis_name="c", subcore_axis_name="s")

  @functools.partial(
      pl.kernel, mesh=mesh,
      compiler_params=pltpu.CompilerParams(use_tc_tiling_on_sc=False),
      out_type=jax.ShapeDtypeStruct((2, mp, wd), jnp.float32),
      scratch_types=[
          pltpu.VMEM((CHUNK,), jnp.int32),
          pltpu.VMEM((CHUNK,), jnp.int32),
          pltpu.VMEM((CHUNK,), jnp.int32),
          pltpu.VMEM((CHUNK,), jnp.int32),
          pltpu.VMEM((CHUNK, wd), jnp.float32),
          pltpu.VMEM((CHUNK, wd), jnp.float32),
          pltpu.VMEM((CHUNK, wd), jnp.float32),
          pltpu.VMEM_SHARED((mp, wd), jnp.float32),
          pltpu.SemaphoreType.DMA,
          pltpu.SemaphoreType.DMA,
      ],
  )
  def k(table, sidx, didx, out, sbuf, dbuf, sbuf2, dbuf2, rows, rows2, zbuf,
        shared, sem, sem2):
    c = lax.axis_index("c")
    s = lax.axis_index("s")
    w = s * 2 + c

    def zrow(i, carry):
      for kk in range(wd // 16):
        zbuf[i, pl.ds(kk * 16, 16)] = jnp.zeros((16,), jnp.float32)
      return carry
    lax.fori_loop(0, CHUNK, zrow, None)

    rbase = s * rpw
    def zcp(j, carry):
      pltpu.sync_copy(zbuf.at[pl.ds(0, cz)],
                      shared.at[pl.ds(rbase + j * cz, cz)])
      return carry
    lax.fori_loop(0, ncz, zcp, None)
    plsc.subcore_barrier()

    ebase = w * per_w
    if ones_src:
      def fill(i, carry):
        for kk in range(wd // 16):
          rows[i, pl.ds(kk * 16, 16)] = jnp.ones((16,), jnp.float32)
        return carry
      lax.fori_loop(0, CHUNK, fill, None)

      def cbody(j, carry):
        pltpu.sync_copy(didx.at[pl.ds(ebase + j * CHUNK, CHUNK)], dbuf)
        pltpu.sync_copy(rows, shared.at[dbuf], add=True)
        return carry
      lax.fori_loop(0, nch, cbody, None)
    else:
      pltpu.sync_copy(sidx.at[pl.ds(ebase, CHUNK)], sbuf)
      pltpu.sync_copy(didx.at[pl.ds(ebase, CHUNK)], dbuf)
      pltpu.async_copy(table.at[sbuf], rows, sem)

      def body2(i, carry):
        o = ebase + (2 * i + 1) * CHUNK
        pltpu.sync_copy(sidx.at[pl.ds(o, CHUNK)], sbuf2)
        pltpu.sync_copy(didx.at[pl.ds(o, CHUNK)], dbuf2)
        pltpu.async_copy(table.at[sbuf2], rows2, sem2)
        pltpu.make_async_copy(table.at[sbuf], rows, sem).wait()
        pltpu.sync_copy(rows, shared.at[dbuf], add=True)

        @pl.when(i < (nch // 2 - 1))
        def _():
          e2 = ebase + (2 * i + 2) * CHUNK
          pltpu.sync_copy(sidx.at[pl.ds(e2, CHUNK)], sbuf)
          pltpu.sync_copy(didx.at[pl.ds(e2, CHUNK)], dbuf)
          pltpu.async_copy(table.at[sbuf], rows, sem)

        pltpu.make_async_copy(table.at[sbuf2], rows2, sem2).wait()
        pltpu.sync_copy(rows2, shared.at[dbuf2], add=True)
        return carry
      lax.fori_loop(0, nch // 2, body2, None)
      if nch % 2 == 1:
        b = ebase + (nch - 1) * CHUNK
        pltpu.sync_copy(sidx.at[pl.ds(b, CHUNK)], sbuf2)
        pltpu.sync_copy(didx.at[pl.ds(b, CHUNK)], dbuf2)
        pltpu.async_copy(table.at[sbuf2], rows2, sem2).wait()
        pltpu.sync_copy(rows2, shared.at[dbuf2], add=True)
    plsc.subcore_barrier()

    def cout(j, carry):
      pltpu.sync_copy(shared.at[pl.ds(rbase + j * cz, cz)],
                      out.at[c, pl.ds(rbase + j * cz, cz)])
      return carry
    lax.fori_loop(0, ncz, cout, None)

  return k


def _mask_body(dref, tref, oref):
  d = dref[...]
  t = tref[...]
  for r in range(NUM_REL):
    oref[r] = jnp.where(t == r, d, SINK_N)


def _masked_dst(dst2, typ2):
  eb = EP // 128
  return pl.pallas_call(
      _mask_body,
      grid=(eb // 8,),
      in_specs=[pl.BlockSpec((8, 128), lambda i: (i, 0)),
                pl.BlockSpec((8, 128), lambda i: (i, 0))],
      out_specs=pl.BlockSpec((NUM_REL, 8, 128), lambda i: (0, i, 0)),
      out_shape=jax.ShapeDtypeStruct((NUM_REL, eb, 128), jnp.int32),
  )(dst2, typ2)


def _embed_body(x0r, x1r, seref, ceref, w0ref, b0ref, oref):
  a = x0r[...]
  b = x1r[...]
  io8 = lax.broadcasted_iota(jnp.int32, (1024, 8), 1)
  s = (a == io8).astype(jnp.float32) @ seref[...]
  col = (b == io8).astype(jnp.float32) @ ceref[...]
  h16 = jnp.concatenate([s, col], axis=1)
  oref[...] = jnp.maximum(h16 @ w0ref[...] + b0ref[...], 0.0)


def _embed(x0, x1, shape_emb, color_emb, w0, b0):
  return pl.pallas_call(
      _embed_body,
      grid=(N2 // 1024,),
      in_specs=[pl.BlockSpec((1024, 1), lambda i: (i, 0)),
                pl.BlockSpec((1024, 1), lambda i: (i, 0)),
                pl.BlockSpec((8, 8), lambda i: (0, 0)),
                pl.BlockSpec((8, 8), lambda i: (0, 0)),
                pl.BlockSpec((16, HID), lambda i: (0, 0)),
                pl.BlockSpec((1, HID), lambda i: (0, 0))],
      out_specs=pl.BlockSpec((1024, HID), lambda i: (i, 0)),
      out_shape=jax.ShapeDtypeStruct((N2, HID), jnp.float32),
  )(x0, x1, shape_emb, color_emb, w0, b0.reshape(1, HID))


def _combine_body(href, sref, cref, wref, wrref, bref, oref):
  acc = href[...] @ wrref[...] + bref[...]
  for r in range(NUM_REL):
    sr = sref[r, 0] + sref[r, 1]
    cr = cref[r, 0] + cref[r, 1]
    acc = acc + (sr @ wref[r]) * (1.0 / jnp.maximum(cr, 1.0))
  oref[...] = jnp.maximum(acc, 0.0)


def _combine(h, s_all, cnt4, w, wr, b):
  return pl.pallas_call(
      _combine_body,
      grid=(N2 // 1024,),
      in_specs=[pl.BlockSpec((1024, HID), lambda i: (i, 0)),
                pl.BlockSpec((NUM_REL, 2, 1024, HID), lambda i: (0, 0, i, 0)),
                pl.BlockSpec((NUM_REL, 2, 1024, 1), lambda i: (0, 0, i, 0)),
                pl.BlockSpec((NUM_REL, HID, HID), lambda i: (0, 0, 0)),
                pl.BlockSpec((HID, HID), lambda i: (0, 0)),
                pl.BlockSpec((1, HID), lambda i: (0, 0))],
      out_specs=pl.BlockSpec((1024, HID), lambda i: (i, 0)),
      out_shape=jax.ShapeDtypeStruct((N2, HID), jnp.float32),
  )(h, s_all, cnt4, w, wr, b.reshape(1, HID))


def _final_body(pref, cref, wcref, bcref, oref):
  ps = pref[0, :NUM_GRAPHS, :] + pref[1, :NUM_GRAPHS, :]
  pc = cref[0, :NUM_GRAPHS, 0:1] + cref[1, :NUM_GRAPHS, 0:1]
  pooled = ps / jnp.maximum(pc, 1.0)
  oref[...] = pooled @ wcref[...] + bcref[...]


def _final(psum, pcnt, wc, bc):
  return pl.pallas_call(
      _final_body,
      out_shape=jax.ShapeDtypeStruct((NUM_GRAPHS, wc.shape[1]), jnp.float32),
  )(psum, pcnt, wc, bc.reshape(1, -1))


def _layer_scatter(h, src_p, dmask_flat):
  ha = h[:, :32]
  hb = h[:, 32:]
  k32 = _sc_scatter(N2, 32, EP, MP_N)
  parts = []
  for r in range(NUM_REL):
    sa = k32(ha, src_p, dmask_flat[r])
    sb = k32(hb, src_p, dmask_flat[r])
    parts.append(jnp.concatenate([sa, sb], axis=-1))
  return jnp.stack(parts)[:, :, :N2, :]


def kernel(x, edge_index, edge_type, batch, shape_emb, color_emb,
           W0, b0, W1, Wr1, b1, W2, Wr2, b2, Wc, bc):
  x = x.astype(jnp.int32)
  src = edge_index[0].astype(jnp.int32)
  dst = edge_index[1].astype(jnp.int32)
  typ = edge_type.astype(jnp.int32)
  batch = batch.astype(jnp.int32)

  x0 = jnp.pad(x[:, 0], (0, N2 - N)).reshape(N2, 1)
  x1 = jnp.pad(x[:, 1], (0, N2 - N)).reshape(N2, 1)
  src_p = jnp.pad(src, (0, EP - E))
  dst_p = jnp.pad(dst, (0, EP - E), constant_values=SINK_N)
  typ_p = jnp.pad(typ, (0, EP - E), constant_values=NUM_REL)

  dmask = _masked_dst(dst_p.reshape(EP // 128, 128),
                      typ_p.reshape(EP // 128, 128))
  dmask_flat = dmask.reshape(NUM_REL, EP)

  ones_tab = jnp.ones((8, 16), jnp.float32)
  zsrc = jnp.zeros((EP,), jnp.int32)
  k16 = _sc_scatter(8, 16, EP, MP_N, True)
  cnts = jnp.stack([k16(ones_tab, zsrc, dmask_flat[r]) for r in range(NUM_REL)])
  cnt4 = cnts[:, :, :N2, 0:1]

  h0 = _embed(x0, x1, shape_emb, color_emb, W0, b0)
  s1 = _layer_scatter(h0, src_p, dmask_flat)
  h1 = _combine(h0, s1, cnt4, W1, Wr1, b1)
  s2 = _layer_scatter(h1, src_p, dmask_flat)
  h2 = _combine(h1, s2, cnt4, W2, Wr2, b2)

  psrc = jnp.pad(jnp.arange(N, dtype=jnp.int32), (0, NP - N))
  pdst = jnp.pad(batch, (0, NP - N), constant_values=SINK_G)
  psum = _sc_scatter(N2, HID, NP, MP_G)(h2, psrc, pdst)
  pcnt = _sc_scatter(8, 16, NP, MP_G, True)(ones_tab, jnp.zeros((NP,), jnp.int32),
                                            pdst)
  return _final(psum, pcnt, Wc, bc)
